# baseline (speedup 1.0000x reference)
import functools

import jax
import jax.numpy as jnp
from jax import lax
from jax.experimental import pallas as pl
from jax.experimental.pallas import tpu as pltpu
from jax.experimental.pallas import tpu_sc as plsc

B = 1024
L = 20
D = 32
TSPAN = 256
NW = 32
BPW = B // NW
PAIRS = L * L
RPW = BPW * PAIRS
TILE = 128
NTILE = RPW // TILE
IPW = BPW * L
ITILE = IPW // TILE
SQRT_D = float(D) ** 0.5

_mesh = plsc.VectorSubcoreMesh(core_axis_name="c", subcore_axis_name="s")


@functools.partial(
    pl.kernel,
    mesh=_mesh,
    compiler_params=pltpu.CompilerParams(needs_layout_passes=False,
                                         use_tc_tiling_on_sc=False),
    out_type=[
        jax.ShapeDtypeStruct((B * L, D), jnp.float32),
        jax.ShapeDtypeStruct((B * PAIRS, D), jnp.float32),
        jax.ShapeDtypeStruct((B * PAIRS, D), jnp.float32),
        jax.ShapeDtypeStruct((B * L, D), jnp.float32),
        jax.ShapeDtypeStruct((B * L, D), jnp.float32),
    ],
    scratch_types=[
        pltpu.VMEM((BPW, L), jnp.int32),
        pltpu.VMEM((IPW,), jnp.int32),
        pltpu.VMEM((IPW,), jnp.float32),
        pltpu.VMEM((NTILE, TILE), jnp.int32),
        pltpu.VMEM((TILE, D), jnp.float32),
        pltpu.VMEM((TILE, D), jnp.float32),
        pltpu.VMEM((TILE, D), jnp.float32),
        pltpu.VMEM((8 * L, D), jnp.float32),
        pltpu.VMEM((8 * L, D), jnp.float32),
        pltpu.SemaphoreType.DMA,
        pltpu.SemaphoreType.DMA,
    ],
)
def _sc_embed(iid_hbm, ts_hbm, scale_hbm, tab_hbm, pk_hbm, pv_hbm, tk_hbm,
              tv_hbm, seqs_out, tmk_out, tmv_out, posk_out, posv_out,
              ts_v, iid_v, scale_v, idx_v, kbuf, vbuf, ibuf, pkv, pvv,
              semk, semv):
    wid = lax.axis_index("s") * 2 + lax.axis_index("c")
    base_b = wid * BPW

    pltpu.sync_copy(ts_hbm.at[pl.ds(base_b, BPW)], ts_v)
    pltpu.sync_copy(iid_hbm.at[pl.ds(wid * IPW, IPW)], iid_v)
    pltpu.sync_copy(scale_hbm.at[pl.ds(wid * IPW, IPW)], scale_v)
    for m in range(8):
        pltpu.sync_copy(pk_hbm, pkv.at[pl.ds(m * L, L)])
        pltpu.sync_copy(pv_hbm, pvv.at[pl.ds(m * L, L)])

    lanes = lax.iota(jnp.int32, 16)

    def blk_body(blk, carry):
        g = blk * 16 + lanes
        b = (g * 5243) >> 21
        p = g - b * PAIRS
        i = (p * 3277) >> 16
        j = p - i * L
        ti = plsc.load_gather(ts_v, [b, i])
        tj = plsc.load_gather(ts_v, [b, j])
        tm = jnp.minimum(jnp.abs(ti - tj), TSPAN)
        row = blk >> 3
        col = (blk & 7) * 16
        idx_v[row, pl.ds(col, 16)] = tm
        return carry

    lax.fori_loop(0, RPW // 16, blk_body, 0)

    def tile_body(t, carry):
        pltpu.async_copy(tk_hbm.at[idx_v.at[t]], kbuf, semk).wait()
        pltpu.async_copy(tv_hbm.at[idx_v.at[t]], vbuf, semv).wait()
        row0 = wid * RPW + t * TILE
        pltpu.sync_copy(kbuf, tmk_out.at[pl.ds(row0, TILE)])
        pltpu.sync_copy(vbuf, tmv_out.at[pl.ds(row0, TILE)])
        return carry

    lax.fori_loop(0, NTILE, tile_body, 0)

    def item_body(k, carry):
        pltpu.async_copy(tab_hbm.at[iid_v.at[pl.ds(k * TILE, TILE)]], ibuf,
                         semk).wait()

        def scale_body(blk, c2):
            r = blk >> 1
            off = (blk & 1) * 16
            s = plsc.load_gather(scale_v, [jnp.full((16,), k * TILE + r,
                                                    jnp.int32)])
            ibuf[r, pl.ds(off, 16)] = ibuf[r, pl.ds(off, 16)] * s
            return c2

        lax.fori_loop(0, TILE * D // 16, scale_body, 0)
        pltpu.sync_copy(ibuf, seqs_out.at[pl.ds(wid * IPW + k * TILE, TILE)])
        return carry

    lax.fori_loop(0, ITILE, item_body, 0)

    def pos_body(m2, carry):
        r0 = wid * IPW + m2 * 8 * L
        pltpu.sync_copy(pkv, posk_out.at[pl.ds(r0, 8 * L)])
        pltpu.sync_copy(pvv, posv_out.at[pl.ds(r0, 8 * L)])
        return carry

    lax.fori_loop(0, BPW // 8, pos_body, 0)


def kernel(item_id, timestamp, padding_mask, item_table, pos_k_table,
           pos_v_table, time_k_table, time_v_table):
    scale = jnp.where(padding_mask.reshape(B * L), SQRT_D, 0.0).astype(
        jnp.float32)
    iid = item_id.reshape(B * L)
    seqs, tmk, tmv, posk, posv = _sc_embed(
        iid, timestamp, scale, item_table, pos_k_table, pos_v_table,
        time_k_table, time_v_table)
    return (seqs.reshape(B, L, D),
            tmk.reshape(B, L, L, D),
            tmv.reshape(B, L, L, D),
            posk.reshape(B, L, D),
            posv.reshape(B, L, D))


# speedup vs baseline: 1.0045x; 1.0045x over previous
import functools

import jax
import jax.numpy as jnp
from jax import lax
from jax.experimental import pallas as pl
from jax.experimental.pallas import tpu as pltpu
from jax.experimental.pallas import tpu_sc as plsc

B = 1024
L = 20
D = 32
TSPAN = 256
NW = 32
BPW = B // NW
PAIRS = L * L
RPW = BPW * PAIRS
TILE = 128
NTILE = RPW // TILE
CH = 4
CROWS = CH * TILE
NCH = NTILE // CH
IPW = BPW * L
ITILE = IPW // TILE
SQRT_D = float(D) ** 0.5

_mesh = plsc.VectorSubcoreMesh(core_axis_name="c", subcore_axis_name="s")


@functools.partial(
    pl.kernel,
    mesh=_mesh,
    compiler_params=pltpu.CompilerParams(needs_layout_passes=False,
                                         use_tc_tiling_on_sc=False),
    out_type=[
        jax.ShapeDtypeStruct((B * L, D), jnp.float32),
        jax.ShapeDtypeStruct((B * PAIRS, D), jnp.float32),
        jax.ShapeDtypeStruct((B * PAIRS, D), jnp.float32),
        jax.ShapeDtypeStruct((B * L, D), jnp.float32),
        jax.ShapeDtypeStruct((B * L, D), jnp.float32),
    ],
    scratch_types=[
        pltpu.VMEM((BPW, L), jnp.int32),
        pltpu.VMEM((IPW,), jnp.int32),
        pltpu.VMEM((IPW,), jnp.float32),
        pltpu.VMEM((NTILE, TILE), jnp.int32),
        pltpu.VMEM((2, CROWS, D), jnp.float32),
        pltpu.VMEM((2, CROWS, D), jnp.float32),
        pltpu.VMEM((IPW, D), jnp.float32),
        pltpu.VMEM((8 * L, D), jnp.float32),
        pltpu.VMEM((8 * L, D), jnp.float32),
        pltpu.SemaphoreType.DMA,
        pltpu.SemaphoreType.DMA,
        pltpu.SemaphoreType.DMA,
        pltpu.SemaphoreType.DMA,
        pltpu.SemaphoreType.DMA,
        pltpu.SemaphoreType.DMA,
    ],
)
def _sc_embed(iid_hbm, ts_hbm, scale_hbm, tab_hbm, pk_hbm, pv_hbm, tk_hbm,
              tv_hbm, seqs_out, tmk_out, tmv_out, posk_out, posv_out,
              ts_v, iid_v, scale_v, idx_v, kbuf, vbuf, ibuf, pkv, pvv,
              gsem0, gsem1, wsem0, wsem1, isem, psem):
    wid = lax.axis_index("s") * 2 + lax.axis_index("c")
    base_b = wid * BPW
    gsem = [gsem0, gsem1]
    wsem = [wsem0, wsem1]

    pltpu.sync_copy(ts_hbm.at[pl.ds(base_b, BPW)], ts_v)
    pltpu.async_copy(iid_hbm.at[pl.ds(wid * IPW, IPW)], iid_v, isem)
    pltpu.async_copy(scale_hbm.at[pl.ds(wid * IPW, IPW)], scale_v, isem)
    for m in range(8):
        pltpu.async_copy(pk_hbm, pkv.at[pl.ds(m * L, L)], psem)
        pltpu.async_copy(pv_hbm, pvv.at[pl.ds(m * L, L)], psem)

    lanes = lax.iota(jnp.int32, 16)

    def blk_body(blk, carry):
        g = blk * 16 + lanes
        b = (g * 5243) >> 21
        p = g - b * PAIRS
        i = (p * 3277) >> 16
        j = p - i * L
        ti = plsc.load_gather(ts_v, [b, i])
        tj = plsc.load_gather(ts_v, [b, j])
        tm = jnp.minimum(jnp.abs(ti - tj), TSPAN)
        row = blk >> 3
        col = (blk & 7) * 16
        idx_v[row, pl.ds(col, 16)] = tm
        return carry

    lax.fori_loop(0, RPW // 16, blk_body, 0)

    pltpu.make_async_copy(iid_hbm.at[pl.ds(0, IPW)], iid_v, isem).wait()
    pltpu.make_async_copy(scale_hbm.at[pl.ds(0, IPW)], scale_v, isem).wait()
    for k in range(ITILE):
        pltpu.async_copy(tab_hbm.at[iid_v.at[pl.ds(k * TILE, TILE)]],
                         ibuf.at[pl.ds(k * TILE, TILE)], isem)

    for m in range(8):
        pltpu.make_async_copy(pk_hbm, pkv.at[pl.ds(m * L, L)], psem).wait()
        pltpu.make_async_copy(pv_hbm, pvv.at[pl.ds(m * L, L)], psem).wait()
    for m in range(BPW // 8):
        r0 = wid * IPW + m * 8 * L
        pltpu.async_copy(pkv, posk_out.at[pl.ds(r0, 8 * L)], psem)
        pltpu.async_copy(pvv, posv_out.at[pl.ds(r0, 8 * L)], psem)

    def fire_gathers(c, s):
        kb, vb = kbuf.at[s], vbuf.at[s]
        for t in range(CH):
            tile = c * CH + t
            pltpu.async_copy(tk_hbm.at[idx_v.at[tile]],
                             kb.at[pl.ds(t * TILE, TILE)], gsem[s])
            pltpu.async_copy(tv_hbm.at[idx_v.at[tile]],
                             vb.at[pl.ds(t * TILE, TILE)], gsem[s])

    def drain_gathers(s):
        pltpu.make_async_copy(tmk_out.at[pl.ds(0, CROWS)], kbuf.at[s],
                              gsem[s]).wait()
        pltpu.make_async_copy(tmk_out.at[pl.ds(0, CROWS)], vbuf.at[s],
                              gsem[s]).wait()

    def drain_writes(s):
        pltpu.make_async_copy(tmk_out.at[pl.ds(0, CROWS)], kbuf.at[s],
                              wsem[s]).wait()
        pltpu.make_async_copy(tmk_out.at[pl.ds(0, CROWS)], vbuf.at[s],
                              wsem[s]).wait()

    def fire_writes(c, s):
        row0 = wid * RPW + c * CROWS
        pltpu.async_copy(kbuf.at[s], tmk_out.at[pl.ds(row0, CROWS)], wsem[s])
        pltpu.async_copy(vbuf.at[s], tmv_out.at[pl.ds(row0, CROWS)], wsem[s])

    fire_gathers(0, 0)

    def chunk_pair(g, carry):
        for par in range(2):
            c = g * 2 + par
            s = par
            s2 = 1 - par
            drain_gathers(s)

            @pl.when(c >= 1)
            def _drain_w():
                drain_writes(s2)

            fire_gathers(c + 1, s2)
            fire_writes(c, s)
        return carry

    lax.fori_loop(0, (NCH - 1) // 2, chunk_pair, 0)
    drain_gathers(0)
    drain_writes(1)
    fire_writes(NCH - 1, 0)

    for k in range(ITILE):
        pltpu.make_async_copy(tab_hbm.at[pl.ds(0, TILE)],
                              ibuf.at[pl.ds(k * TILE, TILE)], isem).wait()

    def scale_body(blk, carry):
        r = blk >> 1
        off = (blk & 1) * 16
        sc = plsc.load_gather(scale_v, [jnp.full((16,), r, jnp.int32)])
        ibuf[r, pl.ds(off, 16)] = ibuf[r, pl.ds(off, 16)] * sc
        return carry

    lax.fori_loop(0, IPW * D // 16, scale_body, 0)
    pltpu.async_copy(ibuf, seqs_out.at[pl.ds(wid * IPW, IPW)], isem)

    drain_writes(0)
    pltpu.make_async_copy(tab_hbm.at[pl.ds(0, IPW)], ibuf, isem).wait()
    for m in range(BPW // 8):
        pltpu.make_async_copy(pkv, posk_out.at[pl.ds(0, 8 * L)], psem).wait()
        pltpu.make_async_copy(pvv, posv_out.at[pl.ds(0, 8 * L)], psem).wait()


def kernel(item_id, timestamp, padding_mask, item_table, pos_k_table,
           pos_v_table, time_k_table, time_v_table):
    scale = jnp.where(padding_mask.reshape(B * L), SQRT_D, 0.0).astype(
        jnp.float32)
    iid = item_id.reshape(B * L)
    seqs, tmk, tmv, posk, posv = _sc_embed(
        iid, timestamp, scale, item_table, pos_k_table, pos_v_table,
        time_k_table, time_v_table)
    return (seqs.reshape(B, L, D),
            tmk.reshape(B, L, L, D),
            tmv.reshape(B, L, L, D),
            posk.reshape(B, L, D),
            posv.reshape(B, L, D))
